# Initial kernel scaffold; baseline (speedup 1.0000x reference)
import jax
import jax.numpy as jnp
from jax.experimental import pallas as pl


def kernel(patch_embeddings, encoder_mask_emb):
    raise NotImplementedError("write your pallas kernel here")


# speedup vs baseline: 2.4231x; 2.4231x over previous
import jax
import jax.numpy as jnp
from jax.experimental import pallas as pl
from jax.experimental.pallas import tpu as pltpu

_MASK_PCT = 0.75
_BB = 4


def _select_body(mask_ref, emb_ref, x_ref, out_ref):
    m = mask_ref[...] != 0
    out_ref[...] = jnp.where(m, emb_ref[...], x_ref[...])


def kernel(patch_embeddings, encoder_mask_emb):
    B, N, D = patch_embeddings.shape
    M = int(_MASK_PCT * N)

    keys = jax.random.split(jax.random.key(42), B)
    perms = jax.vmap(lambda k: jax.random.permutation(k, N))(keys)
    masked_indices = perms[:, :M]
    unmasked_indices = perms[:, M:]

    mask = (
        jnp.zeros((B, N), dtype=jnp.int32)
        .at[jnp.arange(B)[:, None], masked_indices]
        .set(1)
    ).reshape(B, N, 1)
    emb3 = encoder_mask_emb.reshape(1, 1, D)

    bb = _BB
    grid = (B // bb,)
    out = pl.pallas_call(
        _select_body,
        grid=grid,
        in_specs=[
            pl.BlockSpec((bb, N, 1), lambda i: (i, 0, 0)),
            pl.BlockSpec((1, 1, D), lambda i: (0, 0, 0)),
            pl.BlockSpec((bb, N, D), lambda i: (i, 0, 0)),
        ],
        out_specs=pl.BlockSpec((bb, N, D), lambda i: (i, 0, 0)),
        out_shape=jax.ShapeDtypeStruct((B, N, D), patch_embeddings.dtype),
    )(mask, emb3, patch_embeddings)

    return out, masked_indices, unmasked_indices


# speedup vs baseline: 6.6893x; 2.7607x over previous
import functools

import jax
import jax.numpy as jnp
import numpy as np
from jax.experimental import pallas as pl
from jax.experimental.pallas import tpu as pltpu

_MASK_PCT = 0.75
_BB = 4


@functools.lru_cache(maxsize=None)
def _index_plan(B, N):
    M = int(_MASK_PCT * N)

    with jax.ensure_compile_time_eval():
        keys = jax.random.split(jax.random.key(42), B)
        perms = np.asarray(
            jax.vmap(lambda k: jax.random.permutation(k, N))(keys)
        )
    masked = perms[:, :M].astype(np.int32)
    unmasked = perms[:, M:].astype(np.int32)
    mask = np.zeros((B, N, 1), dtype=np.int32)
    np.put_along_axis(mask[:, :, 0], masked, 1, axis=1)
    return masked, unmasked, mask


def _select_body(mask_ref, emb_ref, x_ref, out_ref):
    m = mask_ref[...] != 0
    out_ref[...] = jnp.where(m, emb_ref[...], x_ref[...])


def kernel(patch_embeddings, encoder_mask_emb):
    B, N, D = patch_embeddings.shape

    masked_np, unmasked_np, mask_np = _index_plan(B, N)
    masked_indices = jnp.asarray(masked_np)
    unmasked_indices = jnp.asarray(unmasked_np)
    mask = jnp.asarray(mask_np)
    emb3 = encoder_mask_emb.reshape(1, 1, D)

    bb = _BB
    grid = (B // bb,)
    out = pl.pallas_call(
        _select_body,
        grid=grid,
        in_specs=[
            pl.BlockSpec((bb, N, 1), lambda i: (i, 0, 0)),
            pl.BlockSpec((1, 1, D), lambda i: (0, 0, 0)),
            pl.BlockSpec((bb, N, D), lambda i: (i, 0, 0)),
        ],
        out_specs=pl.BlockSpec((bb, N, D), lambda i: (i, 0, 0)),
        out_shape=jax.ShapeDtypeStruct((B, N, D), patch_embeddings.dtype),
    )(mask, emb3, patch_embeddings)

    return out, masked_indices, unmasked_indices
